# baseline (speedup 1.0000x reference)
import functools

import jax
import jax.numpy as jnp
import numpy as np
from jax import lax
from jax.experimental import pallas as pl
from jax.experimental.pallas import tpu as pltpu
from jax.experimental.pallas import tpu_sc as plsc

_B = 64
_N = 512
_TOK = _B * _N
_D = 256
_CPOS = 128
_NAA = 21
_NTOK = 21
_NMETH = 5
_MAXLEN = 2056.0

_NC = 2
_NS = 16
_NW = _NC * _NS
_PER_W = _TOK // _NW
_CH = 64
_NCHUNK = _PER_W // _CH
_PROWS = 520


def _tables_body(so3_ref, r3_ref, cat_ref, sm_ref, aatable_ref, torw_ref,
                 torb_ref, mtable_ref, wpos_ref, wso3_ref, wr3_ref, waa_ref,
                 wcat_ref, wchain_ref, wtors_ref, wmeth_ref, linb_ref,
                 p_out, ch_out, a_out, wt_out, tvec_out, mvec_out, cvec_out):
    f32 = jnp.float32

    def index_table(nrows, max_len, w_ref, zero_from=None):
        rowi = lax.broadcasted_iota(jnp.int32, (nrows, _CPOS // 2), 0)
        row = rowi.astype(f32)
        k = lax.broadcasted_iota(jnp.int32, (nrows, _CPOS // 2), 1).astype(f32)
        inv_div = jnp.exp(k * (-2.0 * np.log(max_len) / _CPOS))
        ang = row * inv_div
        emb = jnp.concatenate([jnp.sin(ang), jnp.cos(ang)], axis=1)
        if zero_from is not None:
            zmask = (lax.broadcasted_iota(jnp.int32, (nrows, _CPOS), 0)
                     < zero_from).astype(f32)
            emb = emb * zmask
        return jnp.dot(emb, w_ref[...], preferred_element_type=f32)

    p_out[...] = index_table(_PROWS, 2056.0, wpos_ref, zero_from=512)
    ch_out[...] = index_table(128, 100.0, wchain_ref)
    a_out[...] = jnp.dot(aatable_ref[...], waa_ref[...],
                         preferred_element_type=f32)
    wt_out[...] = jnp.dot(torw_ref[...], wtors_ref[...],
                          preferred_element_type=f32)
    cvec_out[...] = linb_ref[...] + jnp.dot(torb_ref[...], wtors_ref[...],
                                            preferred_element_type=f32)

    def time_vec(ts_ref, w_ref):
        t = ts_ref[...] * _MAXLEN
        k = lax.broadcasted_iota(jnp.int32, (_B, 64), 1).astype(f32)
        freqs = jnp.exp(k * (-np.log(_MAXLEN) / 63.0))
        emb = t * freqs
        te = jnp.concatenate([jnp.sin(emb), jnp.cos(emb)], axis=1)
        return jnp.dot(te, w_ref[...], preferred_element_type=f32)

    tvec_out[...] = (time_vec(so3_ref, wso3_ref) + time_vec(r3_ref, wr3_ref)
                     + time_vec(cat_ref, wcat_ref))

    mfold = jnp.dot(mtable_ref[...], wmeth_ref[...],
                    preferred_element_type=f32)
    iota5 = lax.broadcasted_iota(jnp.int32, (_B, _NMETH), 1)
    onehot = (sm_ref[...] == iota5).astype(f32)
    mvec_out[...] = jnp.dot(onehot, mfold, preferred_element_type=f32)


def _build_tables(so3_t, r3_t, cat_t, sm, aatable, torw, torb, mtable,
                  wpos, wso3, wr3, waa, wcat, wchain, wtors, wmeth, linb):
    f32 = jnp.float32
    return pl.pallas_call(
        _tables_body,
        out_shape=[
            jax.ShapeDtypeStruct((_PROWS, _D), f32),
            jax.ShapeDtypeStruct((128, _D), f32),
            jax.ShapeDtypeStruct((_NAA, _D), f32),
            jax.ShapeDtypeStruct((14, _D), f32),
            jax.ShapeDtypeStruct((_B, _D), f32),
            jax.ShapeDtypeStruct((_B, _D), f32),
            jax.ShapeDtypeStruct((1, _D), f32),
        ],
    )(so3_t, r3_t, cat_t, sm, aatable, torw, torb, mtable,
      wpos, wso3, wr3, waa, wcat, wchain, wtors, wmeth, linb)


_NBUF = 6

_SC_SCRATCH = (
    [pltpu.VMEM((_PER_W,), jnp.int32),
     pltpu.VMEM((_PER_W,), jnp.float32)]
    + [pltpu.VMEM((_CH, _D), jnp.float32) for _ in range(_NBUF)]
    + [pltpu.SemaphoreType.DMA for _ in range(2 * _NBUF)]
)


def _sc_body(p_hbm, ri_hbm, mask_hbm, out_hbm, ri_v, mask_v, *bufs_and_sems):
    bufs = bufs_and_sems[:_NBUF]
    gsem = bufs_and_sems[_NBUF:2 * _NBUF]
    wsem = bufs_and_sems[2 * _NBUF:]
    wid = lax.axis_index("s") * _NC + lax.axis_index("c")
    base = wid * _PER_W

    pltpu.sync_copy(ri_hbm.at[pl.ds(base, _PER_W)], ri_v)
    pltpu.sync_copy(mask_hbm.at[pl.ds(base, _PER_W)], mask_v)

    def mask_body(i, mc):
        sl = pl.ds(i * 16, 16)
        ri_v[sl] = jnp.where(mask_v[sl] != 0.0, ri_v[sl], 512)
        return mc

    lax.fori_loop(0, _PER_W // 16, mask_body, 0)

    def gather(c, b):
        return pltpu.make_async_copy(
            p_hbm.at[ri_v.at[pl.ds(c * _CH, _CH)]], bufs[b], gsem[b])

    def writeout(c, b):
        return pltpu.make_async_copy(
            bufs[b], out_hbm.at[pl.ds(base + c * _CH, _CH)], wsem[b])

    for c in range(_NBUF):
        gather(c, c).start()
    for c in range(_NCHUNK):
        b = c % _NBUF
        gather(c, b).wait()
        writeout(c, b).start()
        cp = c - 2
        cf = cp + _NBUF
        if cp >= 0 and cf < _NCHUNK:
            writeout(cp, cp % _NBUF).wait()
            gather(cf, cf % _NBUF).start()
    for c in range(_NCHUNK - _NBUF, _NCHUNK):
        writeout(c, c % _NBUF).wait()


_sc_gather_sum = pl.kernel(
    _sc_body,
    out_type=jax.ShapeDtypeStruct((_TOK, _D), jnp.float32),
    mesh=plsc.VectorSubcoreMesh(core_axis_name="c", subcore_axis_name="s",
                                num_cores=_NC, num_subcores=_NS),
    scratch_types=_SC_SCRATCH,
)


def _combine_body(y1_ref, x_ref, tvec_ref, mvec_ref, cvec_ref, w35_ref,
                  ach_ref, wdm_ref, whot_ref, out_ref):
    f32 = jnp.float32
    x = x_ref[0]
    y = jnp.dot(x[:, 0:35], w35_ref[...], preferred_element_type=f32)
    iota2 = lax.broadcasted_iota(jnp.int32, (_N, _NAA + 100), 1).astype(f32)
    twohot = ((x[:, 38:39] == iota2).astype(f32)
              + (x[:, 39:40] + float(_NAA) == iota2).astype(f32))
    y = y + jnp.dot(twohot, ach_ref[...], preferred_element_type=f32)
    y = y + x[:, 35:36] * wdm_ref[...]
    y = y + x[:, 36:37] * whot_ref[...]
    y = y + x[:, 37:38] * tvec_ref[0]
    y = y + (mvec_ref[0] + cvec_ref[...])
    out_ref[0] = y + y1_ref[0]


def _combine(y1, xfeat, tvec, mvec, cvec, w35, ach, wdm, whot):
    f32 = jnp.float32
    return pl.pallas_call(
        _combine_body,
        grid=(_B,),
        in_specs=[
            pl.BlockSpec((1, _N, _D), lambda b: (b, 0, 0)),
            pl.BlockSpec((1, _N, 40), lambda b: (b, 0, 0)),
            pl.BlockSpec((1, 1, _D), lambda b: (b, 0, 0)),
            pl.BlockSpec((1, 1, _D), lambda b: (b, 0, 0)),
            pl.BlockSpec((1, _D), lambda b: (0, 0)),
            pl.BlockSpec((35, _D), lambda b: (0, 0)),
            pl.BlockSpec((_NAA + 100, _D), lambda b: (0, 0)),
            pl.BlockSpec((1, _D), lambda b: (0, 0)),
            pl.BlockSpec((1, _D), lambda b: (0, 0)),
        ],
        out_specs=pl.BlockSpec((1, _N, _D), lambda b: (b, 0, 0)),
        out_shape=jax.ShapeDtypeStruct((_B, _N, _D), f32),
    )(y1, xfeat, tvec, mvec, cvec, w35, ach, wdm, whot)


def kernel(so3_t, r3_t, cat_t, res_mask, diffuse_mask, chain_index, res_index,
           aatypes, aatypes_sc, torsions_t, structure_method, hot_spots_mask,
           aatype_table, torsion_W, torsion_b, method_table, lin_W, lin_b):
    f32 = jnp.float32
    i32 = jnp.int32

    offs = {}
    cur = 0
    for name, w in [("pos", _CPOS), ("dm", 1), ("so3", 128), ("r3", 128),
                    ("aa", _D), ("cat", 128), ("sc", _NTOK), ("chain", _CPOS),
                    ("tors", 128), ("meth", 128), ("hot", 1)]:
        offs[name] = (cur, cur + w)
        cur += w

    def wb(name):
        s, e = offs[name]
        return lin_W[s:e]

    p_tab, ch_tab, a_tab, wt_fold, tvec, mvec, cvec = _build_tables(
        so3_t.astype(f32), r3_t.astype(f32), cat_t.astype(f32),
        structure_method.astype(i32), aatype_table.astype(f32),
        torsion_W.astype(f32), torsion_b.reshape(1, 128).astype(f32),
        method_table.astype(f32),
        wb("pos"), wb("so3"), wb("r3"), wb("aa"), wb("cat"), wb("chain"),
        wb("tors"), wb("meth"), lin_b.reshape(1, _D).astype(f32))

    y1 = _sc_gather_sum(
        p_tab,
        res_index.reshape(_TOK).astype(i32),
        res_mask.reshape(_TOK).astype(f32))

    w35 = jnp.concatenate([wb("sc"), wt_fold], axis=0)
    ach = jnp.concatenate([a_tab, ch_tab[:100]], axis=0)
    xfeat = jnp.concatenate(
        [aatypes_sc.astype(f32),
         torsions_t.reshape(_B, _N, 14).astype(f32),
         diffuse_mask.reshape(_B, _N, 1).astype(f32),
         hot_spots_mask.astype(f32).reshape(_B, _N, 1),
         res_mask.reshape(_B, _N, 1).astype(f32),
         aatypes.astype(f32).reshape(_B, _N, 1),
         chain_index.astype(f32).reshape(_B, _N, 1)],
        axis=-1)

    out = _combine(
        y1.reshape(_B, _N, _D),
        xfeat,
        tvec.reshape(_B, 1, _D),
        mvec.reshape(_B, 1, _D),
        cvec,
        w35, ach, wb("dm"), wb("hot"))
    return out


# speedup vs baseline: 1.3384x; 1.3384x over previous
import functools

import jax
import jax.numpy as jnp
import numpy as np
from jax import lax
from jax.experimental import pallas as pl
from jax.experimental.pallas import tpu as pltpu
from jax.experimental.pallas import tpu_sc as plsc

_B = 64
_N = 512
_TOK = _B * _N
_D = 256
_CPOS = 128
_NAA = 21
_NTOK = 21
_NMETH = 5
_MAXLEN = 2056.0

_NC = 2
_NS = 16
_NW = _NC * _NS
_PER_W = _TOK // _NW
_CH = 64
_NCHUNK = _PER_W // _CH
_PROWS = 520


def _tables_body(so3_ref, r3_ref, cat_ref, sm_ref, aatable_ref, torw_ref,
                 torb_ref, mtable_ref, wpos_ref, wso3_ref, wr3_ref, waa_ref,
                 wcat_ref, wchain_ref, wtors_ref, wmeth_ref, linb_ref,
                 p_out, ch_out, a_out, wt_out, tvec_out, mvec_out, cvec_out):
    f32 = jnp.float32

    def index_table(nrows, max_len, w_ref, zero_from=None):
        rowi = lax.broadcasted_iota(jnp.int32, (nrows, _CPOS // 2), 0)
        row = rowi.astype(f32)
        k = lax.broadcasted_iota(jnp.int32, (nrows, _CPOS // 2), 1).astype(f32)
        inv_div = jnp.exp(k * (-2.0 * np.log(max_len) / _CPOS))
        ang = row * inv_div
        emb = jnp.concatenate([jnp.sin(ang), jnp.cos(ang)], axis=1)
        if zero_from is not None:
            zmask = (lax.broadcasted_iota(jnp.int32, (nrows, _CPOS), 0)
                     < zero_from).astype(f32)
            emb = emb * zmask
        return jnp.dot(emb, w_ref[...], preferred_element_type=f32)

    p_out[...] = index_table(_PROWS, 2056.0, wpos_ref, zero_from=512)
    ch_out[...] = index_table(128, 100.0, wchain_ref)
    a_out[...] = jnp.dot(aatable_ref[...], waa_ref[...],
                         preferred_element_type=f32)
    wt_out[...] = jnp.dot(torw_ref[...], wtors_ref[...],
                          preferred_element_type=f32)
    cvec_out[...] = linb_ref[...] + jnp.dot(torb_ref[...], wtors_ref[...],
                                            preferred_element_type=f32)

    def time_vec(ts_ref, w_ref):
        t = ts_ref[...] * _MAXLEN
        k = lax.broadcasted_iota(jnp.int32, (_B, 64), 1).astype(f32)
        freqs = jnp.exp(k * (-np.log(_MAXLEN) / 63.0))
        emb = t * freqs
        te = jnp.concatenate([jnp.sin(emb), jnp.cos(emb)], axis=1)
        return jnp.dot(te, w_ref[...], preferred_element_type=f32)

    tvec_out[...] = (time_vec(so3_ref, wso3_ref) + time_vec(r3_ref, wr3_ref)
                     + time_vec(cat_ref, wcat_ref))

    mfold = jnp.dot(mtable_ref[...], wmeth_ref[...],
                    preferred_element_type=f32)
    iota5 = lax.broadcasted_iota(jnp.int32, (_B, _NMETH), 1)
    onehot = (sm_ref[...] == iota5).astype(f32)
    mvec_out[...] = jnp.dot(onehot, mfold, preferred_element_type=f32)


def _build_tables(so3_t, r3_t, cat_t, sm, aatable, torw, torb, mtable,
                  wpos, wso3, wr3, waa, wcat, wchain, wtors, wmeth, linb):
    f32 = jnp.float32
    return pl.pallas_call(
        _tables_body,
        out_shape=[
            jax.ShapeDtypeStruct((_PROWS, _D), f32),
            jax.ShapeDtypeStruct((128, _D), f32),
            jax.ShapeDtypeStruct((_NAA, _D), f32),
            jax.ShapeDtypeStruct((14, _D), f32),
            jax.ShapeDtypeStruct((_B, _D), f32),
            jax.ShapeDtypeStruct((_B, _D), f32),
            jax.ShapeDtypeStruct((1, _D), f32),
        ],
    )(so3_t, r3_t, cat_t, sm, aatable, torw, torb, mtable,
      wpos, wso3, wr3, waa, wcat, wchain, wtors, wmeth, linb)


_NBUF = 7

_SC_SCRATCH = (
    [pltpu.VMEM((_PER_W,), jnp.int32),
     pltpu.VMEM((_PER_W,), jnp.float32)]
    + [pltpu.VMEM((_CH, _D), jnp.float32) for _ in range(_NBUF)]
    + [pltpu.SemaphoreType.DMA for _ in range(2 * _NBUF)]
)


def _sc_body(p_hbm, ri_hbm, mask_hbm, out_hbm, ri_v, mask_v, *bufs_and_sems):
    bufs = bufs_and_sems[:_NBUF]
    gsem = bufs_and_sems[_NBUF:2 * _NBUF]
    wsem = bufs_and_sems[2 * _NBUF:]
    wid = lax.axis_index("s") * _NC + lax.axis_index("c")
    base = wid * _PER_W

    pltpu.sync_copy(ri_hbm.at[pl.ds(base, _PER_W)], ri_v)
    pltpu.sync_copy(mask_hbm.at[pl.ds(base, _PER_W)], mask_v)

    def mask_body(i, mc):
        sl = pl.ds(i * 16, 16)
        ri_v[sl] = jnp.where(mask_v[sl] != 0.0, ri_v[sl], 512)
        return mc

    lax.fori_loop(0, _PER_W // 16, mask_body, 0)

    def gather(c, b):
        return pltpu.make_async_copy(
            p_hbm.at[ri_v.at[pl.ds(c * _CH, _CH)]], bufs[b], gsem[b])

    def writeout(c, b):
        return pltpu.make_async_copy(
            bufs[b], out_hbm.at[pl.ds(base + c * _CH, _CH)], wsem[b])

    for c in range(_NBUF):
        gather(c, c).start()
    for c in range(_NCHUNK):
        b = c % _NBUF
        gather(c, b).wait()
        writeout(c, b).start()
        cp = c - 2
        cf = cp + _NBUF
        if cp >= 0 and cf < _NCHUNK:
            writeout(cp, cp % _NBUF).wait()
            gather(cf, cf % _NBUF).start()
    for c in range(_NCHUNK - _NBUF, _NCHUNK):
        writeout(c, c % _NBUF).wait()


_sc_gather_sum = pl.kernel(
    _sc_body,
    out_type=jax.ShapeDtypeStruct((_TOK, _D), jnp.float32),
    mesh=plsc.VectorSubcoreMesh(core_axis_name="c", subcore_axis_name="s",
                                num_cores=_NC, num_subcores=_NS),
    scratch_types=_SC_SCRATCH,
)


def _combine_body(y1_ref, sctors_ref, aa_ref, ch_ref, dm_ref, hot_ref,
                  mask_ref, tvec_ref, mvec_ref, cvec_ref, w35_ref, ach_ref,
                  wdm_ref, whot_ref, out_ref):
    f32 = jnp.float32
    y = jnp.dot(sctors_ref[0], w35_ref[...], preferred_element_type=f32)
    iota2 = lax.broadcasted_iota(jnp.int32, (_N, _NAA + 100), 1)
    twohot = ((aa_ref[0] == iota2).astype(f32)
              + (ch_ref[0] + _NAA == iota2).astype(f32))
    y = y + jnp.dot(twohot, ach_ref[...], preferred_element_type=f32)
    y = y + dm_ref[0] * wdm_ref[...]
    y = y + hot_ref[0] * whot_ref[...]
    y = y + mask_ref[0] * tvec_ref[0]
    y = y + (mvec_ref[0] + cvec_ref[...])
    out_ref[0] = y + y1_ref[0]


def _combine(y1, sctors, aa, ch, dm, hot, mask, tvec, mvec, cvec,
             w35, ach, wdm, whot):
    f32 = jnp.float32
    return pl.pallas_call(
        _combine_body,
        grid=(_B,),
        in_specs=[
            pl.BlockSpec((1, _N, _D), lambda b: (b, 0, 0)),
            pl.BlockSpec((1, _N, 35), lambda b: (b, 0, 0)),
            pl.BlockSpec((1, _N, 1), lambda b: (b, 0, 0)),
            pl.BlockSpec((1, _N, 1), lambda b: (b, 0, 0)),
            pl.BlockSpec((1, _N, 1), lambda b: (b, 0, 0)),
            pl.BlockSpec((1, _N, 1), lambda b: (b, 0, 0)),
            pl.BlockSpec((1, _N, 1), lambda b: (b, 0, 0)),
            pl.BlockSpec((1, 1, _D), lambda b: (b, 0, 0)),
            pl.BlockSpec((1, 1, _D), lambda b: (b, 0, 0)),
            pl.BlockSpec((1, _D), lambda b: (0, 0)),
            pl.BlockSpec((35, _D), lambda b: (0, 0)),
            pl.BlockSpec((_NAA + 100, _D), lambda b: (0, 0)),
            pl.BlockSpec((1, _D), lambda b: (0, 0)),
            pl.BlockSpec((1, _D), lambda b: (0, 0)),
        ],
        out_specs=pl.BlockSpec((1, _N, _D), lambda b: (b, 0, 0)),
        out_shape=jax.ShapeDtypeStruct((_B, _N, _D), f32),
    )(y1, sctors, aa, ch, dm, hot, mask, tvec, mvec, cvec,
      w35, ach, wdm, whot)


def kernel(so3_t, r3_t, cat_t, res_mask, diffuse_mask, chain_index, res_index,
           aatypes, aatypes_sc, torsions_t, structure_method, hot_spots_mask,
           aatype_table, torsion_W, torsion_b, method_table, lin_W, lin_b):
    f32 = jnp.float32
    i32 = jnp.int32

    offs = {}
    cur = 0
    for name, w in [("pos", _CPOS), ("dm", 1), ("so3", 128), ("r3", 128),
                    ("aa", _D), ("cat", 128), ("sc", _NTOK), ("chain", _CPOS),
                    ("tors", 128), ("meth", 128), ("hot", 1)]:
        offs[name] = (cur, cur + w)
        cur += w

    def wb(name):
        s, e = offs[name]
        return lin_W[s:e]

    p_tab, ch_tab, a_tab, wt_fold, tvec, mvec, cvec = _build_tables(
        so3_t.astype(f32), r3_t.astype(f32), cat_t.astype(f32),
        structure_method.astype(i32), aatype_table.astype(f32),
        torsion_W.astype(f32), torsion_b.reshape(1, 128).astype(f32),
        method_table.astype(f32),
        wb("pos"), wb("so3"), wb("r3"), wb("aa"), wb("cat"), wb("chain"),
        wb("tors"), wb("meth"), lin_b.reshape(1, _D).astype(f32))

    y1 = _sc_gather_sum(
        p_tab,
        res_index.reshape(_TOK).astype(i32),
        res_mask.reshape(_TOK).astype(f32))

    w35 = jnp.concatenate([wb("sc"), wt_fold], axis=0)
    ach = jnp.concatenate([a_tab, ch_tab[:100]], axis=0)
    sctors = jnp.concatenate(
        [aatypes_sc.astype(f32), torsions_t.reshape(_B, _N, 14).astype(f32)],
        axis=-1)

    out = _combine(
        y1.reshape(_B, _N, _D),
        sctors,
        aatypes.reshape(_B, _N, 1).astype(i32),
        chain_index.reshape(_B, _N, 1).astype(i32),
        diffuse_mask.reshape(_B, _N, 1).astype(f32),
        hot_spots_mask.astype(f32).reshape(_B, _N, 1),
        res_mask.reshape(_B, _N, 1).astype(f32),
        tvec.reshape(_B, 1, _D),
        mvec.reshape(_B, 1, _D),
        cvec,
        w35, ach, wb("dm"), wb("hot"))
    return out


# speedup vs baseline: 1.4092x; 1.0529x over previous
import functools

import jax
import jax.numpy as jnp
import numpy as np
from jax import lax
from jax.experimental import pallas as pl
from jax.experimental.pallas import tpu as pltpu
from jax.experimental.pallas import tpu_sc as plsc

_B = 64
_N = 512
_TOK = _B * _N
_D = 256
_CPOS = 128
_NAA = 21
_NTOK = 21
_NMETH = 5
_MAXLEN = 2056.0

_NC = 2
_NS = 16
_NW = _NC * _NS
_PER_W = _TOK // _NW
_CH = 64
_NCHUNK = _PER_W // _CH
_PROWS = 520


def _tables_body(so3_ref, r3_ref, cat_ref, sm_ref, aatable_ref, torw_ref,
                 torb_ref, mtable_ref, wpos_ref, wso3_ref, wr3_ref, waa_ref,
                 wcat_ref, wchain_ref, wtors_ref, wmeth_ref, linb_ref,
                 p_out, ch_out, a_out, wt_out, tvec_out, mvec_out, cvec_out):
    f32 = jnp.float32

    def index_table(nrows, max_len, w_ref, zero_from=None):
        rowi = lax.broadcasted_iota(jnp.int32, (nrows, _CPOS // 2), 0)
        row = rowi.astype(f32)
        k = lax.broadcasted_iota(jnp.int32, (nrows, _CPOS // 2), 1).astype(f32)
        inv_div = jnp.exp(k * (-2.0 * np.log(max_len) / _CPOS))
        ang = row * inv_div
        emb = jnp.concatenate([jnp.sin(ang), jnp.cos(ang)], axis=1)
        if zero_from is not None:
            zmask = (lax.broadcasted_iota(jnp.int32, (nrows, _CPOS), 0)
                     < zero_from).astype(f32)
            emb = emb * zmask
        return jnp.dot(emb, w_ref[...], preferred_element_type=f32)

    p_out[...] = index_table(_PROWS, 2056.0, wpos_ref, zero_from=512)
    ch_out[...] = index_table(128, 100.0, wchain_ref)
    a_out[...] = jnp.dot(aatable_ref[...], waa_ref[...],
                         preferred_element_type=f32)
    wt_out[...] = jnp.dot(torw_ref[...], wtors_ref[...],
                          preferred_element_type=f32)
    cvec_out[...] = linb_ref[...] + jnp.dot(torb_ref[...], wtors_ref[...],
                                            preferred_element_type=f32)

    def time_vec(ts_ref, w_ref):
        t = ts_ref[...] * _MAXLEN
        k = lax.broadcasted_iota(jnp.int32, (_B, 64), 1).astype(f32)
        freqs = jnp.exp(k * (-np.log(_MAXLEN) / 63.0))
        emb = t * freqs
        te = jnp.concatenate([jnp.sin(emb), jnp.cos(emb)], axis=1)
        return jnp.dot(te, w_ref[...], preferred_element_type=f32)

    tvec_out[...] = (time_vec(so3_ref, wso3_ref) + time_vec(r3_ref, wr3_ref)
                     + time_vec(cat_ref, wcat_ref))

    mfold = jnp.dot(mtable_ref[...], wmeth_ref[...],
                    preferred_element_type=f32)
    iota5 = lax.broadcasted_iota(jnp.int32, (_B, _NMETH), 1)
    onehot = (sm_ref[...] == iota5).astype(f32)
    mvec_out[...] = jnp.dot(onehot, mfold, preferred_element_type=f32)


def _build_tables(so3_t, r3_t, cat_t, sm, aatable, torw, torb, mtable,
                  wpos, wso3, wr3, waa, wcat, wchain, wtors, wmeth, linb):
    f32 = jnp.float32
    return pl.pallas_call(
        _tables_body,
        out_shape=[
            jax.ShapeDtypeStruct((_PROWS, _D), f32),
            jax.ShapeDtypeStruct((128, _D), f32),
            jax.ShapeDtypeStruct((_NAA, _D), f32),
            jax.ShapeDtypeStruct((14, _D), f32),
            jax.ShapeDtypeStruct((_B, _D), f32),
            jax.ShapeDtypeStruct((_B, _D), f32),
            jax.ShapeDtypeStruct((1, _D), f32),
        ],
    )(so3_t, r3_t, cat_t, sm, aatable, torw, torb, mtable,
      wpos, wso3, wr3, waa, wcat, wchain, wtors, wmeth, linb)


_NBUF = 7

_SC_SCRATCH = (
    [pltpu.VMEM((_PER_W,), jnp.int32),
     pltpu.VMEM((_PER_W,), jnp.float32)]
    + [pltpu.VMEM((_CH, _D), jnp.float32) for _ in range(_NBUF)]
    + [pltpu.SemaphoreType.DMA for _ in range(2 * _NBUF)]
)


def _sc_body(p_hbm, ri_hbm, mask_hbm, out_hbm, ri_v, mask_v, *bufs_and_sems):
    bufs = bufs_and_sems[:_NBUF]
    gsem = bufs_and_sems[_NBUF:2 * _NBUF]
    wsem = bufs_and_sems[2 * _NBUF:]
    wid = lax.axis_index("s") * _NC + lax.axis_index("c")
    base = wid * _PER_W

    pltpu.sync_copy(ri_hbm.at[pl.ds(base, _PER_W)], ri_v)
    pltpu.sync_copy(mask_hbm.at[pl.ds(base, _PER_W)], mask_v)

    def mask_body(i, mc):
        sl = pl.ds(i * 16, 16)
        ri_v[sl] = jnp.where(mask_v[sl] != 0.0, ri_v[sl], 512)
        return mc

    lax.fori_loop(0, _PER_W // 16, mask_body, 0)

    def gather(c, b):
        return pltpu.make_async_copy(
            p_hbm.at[ri_v.at[pl.ds(c * _CH, _CH)]], bufs[b], gsem[b])

    def writeout(c, b):
        return pltpu.make_async_copy(
            bufs[b], out_hbm.at[pl.ds(base + c * _CH, _CH)], wsem[b])

    for c in range(_NBUF):
        gather(c, c).start()
    for c in range(_NCHUNK):
        b = c % _NBUF
        gather(c, b).wait()
        writeout(c, b).start()
        cp = c - 2
        cf = cp + _NBUF
        if cp >= 0 and cf < _NCHUNK:
            writeout(cp, cp % _NBUF).wait()
            gather(cf, cf % _NBUF).start()
    for c in range(_NCHUNK - _NBUF, _NCHUNK):
        writeout(c, c % _NBUF).wait()


_sc_gather_sum = pl.kernel(
    _sc_body,
    out_type=jax.ShapeDtypeStruct((_TOK, _D), jnp.float32),
    mesh=plsc.VectorSubcoreMesh(core_axis_name="c", subcore_axis_name="s",
                                num_cores=_NC, num_subcores=_NS),
    scratch_types=_SC_SCRATCH,
)


_NB = 8


def _combine_body(y1_ref, sc_ref, tors_ref, aa_ref, ch_ref, dm_ref, hot_ref,
                  mask_ref, tvec_ref, mvec_ref, cvec_ref, wsc_ref, wt_ref,
                  ach_ref, wdm_ref, whot_ref, out_ref):
    f32 = jnp.float32
    m = _NB * _N
    y = jnp.dot(sc_ref[...].reshape(m, _NTOK), wsc_ref[...],
                preferred_element_type=f32)
    y = y + jnp.dot(tors_ref[...].reshape(m, 14), wt_ref[...],
                    preferred_element_type=f32)
    iota2 = lax.broadcasted_iota(jnp.int32, (m, _NAA + 100), 1)
    twohot = ((aa_ref[...].reshape(m, 1) == iota2).astype(f32)
              + (ch_ref[...].reshape(m, 1) + _NAA == iota2).astype(f32))
    y = y + jnp.dot(twohot, ach_ref[...], preferred_element_type=f32)
    y3 = y.reshape(_NB, _N, _D)
    y3 = y3 + dm_ref[...] * wdm_ref[...][None]
    y3 = y3 + hot_ref[...] * whot_ref[...][None]
    y3 = y3 + mask_ref[...] * tvec_ref[...]
    y3 = y3 + (mvec_ref[...] + cvec_ref[...][None])
    out_ref[...] = y3 + y1_ref[...]


def _combine(y1, sc, tors, aa, ch, dm, hot, mask, tvec, mvec, cvec,
             wsc, wt, ach, wdm, whot):
    f32 = jnp.float32
    return pl.pallas_call(
        _combine_body,
        grid=(_B // _NB,),
        in_specs=[
            pl.BlockSpec((_NB, _N, _D), lambda b: (b, 0, 0)),
            pl.BlockSpec((_NB, _N, _NTOK), lambda b: (b, 0, 0)),
            pl.BlockSpec((_NB, _N, 14), lambda b: (b, 0, 0)),
            pl.BlockSpec((_NB, _N, 1), lambda b: (b, 0, 0)),
            pl.BlockSpec((_NB, _N, 1), lambda b: (b, 0, 0)),
            pl.BlockSpec((_NB, _N, 1), lambda b: (b, 0, 0)),
            pl.BlockSpec((_NB, _N, 1), lambda b: (b, 0, 0)),
            pl.BlockSpec((_NB, _N, 1), lambda b: (b, 0, 0)),
            pl.BlockSpec((_NB, 1, _D), lambda b: (b, 0, 0)),
            pl.BlockSpec((_NB, 1, _D), lambda b: (b, 0, 0)),
            pl.BlockSpec((1, _D), lambda b: (0, 0)),
            pl.BlockSpec((_NTOK, _D), lambda b: (0, 0)),
            pl.BlockSpec((14, _D), lambda b: (0, 0)),
            pl.BlockSpec((_NAA + 100, _D), lambda b: (0, 0)),
            pl.BlockSpec((1, _D), lambda b: (0, 0)),
            pl.BlockSpec((1, _D), lambda b: (0, 0)),
        ],
        out_specs=pl.BlockSpec((_NB, _N, _D), lambda b: (b, 0, 0)),
        out_shape=jax.ShapeDtypeStruct((_B, _N, _D), f32),
    )(y1, sc, tors, aa, ch, dm, hot, mask, tvec, mvec, cvec,
      wsc, wt, ach, wdm, whot)


def kernel(so3_t, r3_t, cat_t, res_mask, diffuse_mask, chain_index, res_index,
           aatypes, aatypes_sc, torsions_t, structure_method, hot_spots_mask,
           aatype_table, torsion_W, torsion_b, method_table, lin_W, lin_b):
    f32 = jnp.float32
    i32 = jnp.int32

    offs = {}
    cur = 0
    for name, w in [("pos", _CPOS), ("dm", 1), ("so3", 128), ("r3", 128),
                    ("aa", _D), ("cat", 128), ("sc", _NTOK), ("chain", _CPOS),
                    ("tors", 128), ("meth", 128), ("hot", 1)]:
        offs[name] = (cur, cur + w)
        cur += w

    def wb(name):
        s, e = offs[name]
        return lin_W[s:e]

    p_tab, ch_tab, a_tab, wt_fold, tvec, mvec, cvec = _build_tables(
        so3_t.astype(f32), r3_t.astype(f32), cat_t.astype(f32),
        structure_method.astype(i32), aatype_table.astype(f32),
        torsion_W.astype(f32), torsion_b.reshape(1, 128).astype(f32),
        method_table.astype(f32),
        wb("pos"), wb("so3"), wb("r3"), wb("aa"), wb("cat"), wb("chain"),
        wb("tors"), wb("meth"), lin_b.reshape(1, _D).astype(f32))

    y1 = _sc_gather_sum(
        p_tab,
        res_index.reshape(_TOK).astype(i32),
        res_mask.reshape(_TOK).astype(f32))

    ach = jnp.concatenate([a_tab, ch_tab[:100]], axis=0)

    out = _combine(
        y1.reshape(_B, _N, _D),
        aatypes_sc.astype(f32),
        torsions_t.reshape(_B, _N, 14).astype(f32),
        aatypes.reshape(_B, _N, 1).astype(i32),
        chain_index.reshape(_B, _N, 1).astype(i32),
        diffuse_mask.reshape(_B, _N, 1).astype(f32),
        hot_spots_mask.astype(f32).reshape(_B, _N, 1),
        res_mask.reshape(_B, _N, 1).astype(f32),
        tvec.reshape(_B, 1, _D),
        mvec.reshape(_B, 1, _D),
        cvec,
        wb("sc"), wt_fold, ach, wb("dm"), wb("hot"))
    return out


# speedup vs baseline: 2.0577x; 1.4602x over previous
import functools

import jax
import jax.numpy as jnp
import numpy as np
from jax import lax
from jax.experimental import pallas as pl
from jax.experimental.pallas import tpu as pltpu
from jax.experimental.pallas import tpu_sc as plsc

_B = 64
_N = 512
_TOK = _B * _N
_D = 256
_CPOS = 128
_NAA = 21
_NTOK = 21
_NMETH = 5
_MAXLEN = 2056.0

_NC = 2
_NS = 16
_NW = _NC * _NS
_PER_W = _TOK // _NW
_CH = 64
_NCHUNK = _PER_W // _CH
_PROWS = 520


def _tables_body(so3_ref, r3_ref, cat_ref, sm_ref, aatable_ref, torw_ref,
                 torb_ref, mtable_ref, wpos_ref, wso3_ref, wr3_ref, waa_ref,
                 wcat_ref, wchain_ref, wtors_ref, wmeth_ref, linb_ref,
                 p_out, ch_out, a_out, wt_out, tvec_out, mvec_out, cvec_out):
    f32 = jnp.float32

    def index_table(nrows, max_len, w_ref, zero_from=None):
        rowi = lax.broadcasted_iota(jnp.int32, (nrows, _CPOS // 2), 0)
        row = rowi.astype(f32)
        k = lax.broadcasted_iota(jnp.int32, (nrows, _CPOS // 2), 1).astype(f32)
        inv_div = jnp.exp(k * (-2.0 * np.log(max_len) / _CPOS))
        ang = row * inv_div
        emb = jnp.concatenate([jnp.sin(ang), jnp.cos(ang)], axis=1)
        if zero_from is not None:
            zmask = (lax.broadcasted_iota(jnp.int32, (nrows, _CPOS), 0)
                     < zero_from).astype(f32)
            emb = emb * zmask
        return jnp.dot(emb, w_ref[...], preferred_element_type=f32)

    p_out[...] = index_table(_PROWS, 2056.0, wpos_ref, zero_from=512)
    ch_out[...] = index_table(128, 100.0, wchain_ref)
    a_out[...] = jnp.dot(aatable_ref[...], waa_ref[...],
                         preferred_element_type=f32)
    wt_out[...] = jnp.dot(torw_ref[...], wtors_ref[...],
                          preferred_element_type=f32)
    cvec_out[...] = linb_ref[...] + jnp.dot(torb_ref[...], wtors_ref[...],
                                            preferred_element_type=f32)

    def time_vec(ts_ref, w_ref):
        t = ts_ref[...] * _MAXLEN
        k = lax.broadcasted_iota(jnp.int32, (_B, 64), 1).astype(f32)
        freqs = jnp.exp(k * (-np.log(_MAXLEN) / 63.0))
        emb = t * freqs
        te = jnp.concatenate([jnp.sin(emb), jnp.cos(emb)], axis=1)
        return jnp.dot(te, w_ref[...], preferred_element_type=f32)

    tvec_out[...] = (time_vec(so3_ref, wso3_ref) + time_vec(r3_ref, wr3_ref)
                     + time_vec(cat_ref, wcat_ref))

    mfold = jnp.dot(mtable_ref[...], wmeth_ref[...],
                    preferred_element_type=f32)
    iota5 = lax.broadcasted_iota(jnp.int32, (_B, _NMETH), 1)
    onehot = (sm_ref[...] == iota5).astype(f32)
    mvec_out[...] = jnp.dot(onehot, mfold, preferred_element_type=f32)


def _build_tables(so3_t, r3_t, cat_t, sm, aatable, torw, torb, mtable,
                  wpos, wso3, wr3, waa, wcat, wchain, wtors, wmeth, linb):
    f32 = jnp.float32
    return pl.pallas_call(
        _tables_body,
        out_shape=[
            jax.ShapeDtypeStruct((_PROWS, _D), f32),
            jax.ShapeDtypeStruct((128, _D), f32),
            jax.ShapeDtypeStruct((_NAA, _D), f32),
            jax.ShapeDtypeStruct((14, _D), f32),
            jax.ShapeDtypeStruct((_B, _D), f32),
            jax.ShapeDtypeStruct((_B, _D), f32),
            jax.ShapeDtypeStruct((1, _D), f32),
        ],
    )(so3_t, r3_t, cat_t, sm, aatable, torw, torb, mtable,
      wpos, wso3, wr3, waa, wcat, wchain, wtors, wmeth, linb)


_NBUF = 7

_SC_SCRATCH = (
    [pltpu.VMEM((_PER_W,), jnp.int32),
     pltpu.VMEM((_PER_W,), jnp.float32)]
    + [pltpu.VMEM((_CH, _D), jnp.float32) for _ in range(_NBUF)]
    + [pltpu.SemaphoreType.DMA for _ in range(2 * _NBUF)]
)


def _sc_body(p_hbm, ri_hbm, mask_hbm, out_hbm, ri_v, mask_v, *bufs_and_sems):
    bufs = bufs_and_sems[:_NBUF]
    gsem = bufs_and_sems[_NBUF:2 * _NBUF]
    wsem = bufs_and_sems[2 * _NBUF:]
    wid = lax.axis_index("s") * _NC + lax.axis_index("c")
    base = wid * _PER_W

    pltpu.sync_copy(ri_hbm.at[pl.ds(base, _PER_W)], ri_v)
    pltpu.sync_copy(mask_hbm.at[pl.ds(base, _PER_W)], mask_v)

    def mask_body(i, mc):
        sl = pl.ds(i * 16, 16)
        ri_v[sl] = jnp.where(mask_v[sl] != 0.0, ri_v[sl], 512)
        return mc

    lax.fori_loop(0, _PER_W // 16, mask_body, 0)

    def gather(c, b):
        return pltpu.make_async_copy(
            p_hbm.at[ri_v.at[pl.ds(c * _CH, _CH)]], bufs[b], gsem[b])

    def writeout(c, b):
        return pltpu.make_async_copy(
            bufs[b], out_hbm.at[pl.ds(base + c * _CH, _CH)], wsem[b])

    for c in range(_NBUF):
        gather(c, c).start()
    for c in range(_NCHUNK):
        b = c % _NBUF
        gather(c, b).wait()
        writeout(c, b).start()
        cp = c - 2
        cf = cp + _NBUF
        if cp >= 0 and cf < _NCHUNK:
            writeout(cp, cp % _NBUF).wait()
            gather(cf, cf % _NBUF).start()
    for c in range(_NCHUNK - _NBUF, _NCHUNK):
        writeout(c, c % _NBUF).wait()


_sc_gather_sum = pl.kernel(
    _sc_body,
    out_type=jax.ShapeDtypeStruct((_TOK, _D), jnp.float32),
    mesh=plsc.VectorSubcoreMesh(core_axis_name="c", subcore_axis_name="s",
                                num_cores=_NC, num_subcores=_NS),
    scratch_types=_SC_SCRATCH,
)


_NB = 8


def _combine_body(y1_ref, sc_ref, tors_ref, aa_ref, ch_ref, dm_ref, hot_ref,
                  mask_ref, tvec_ref, mvec_ref, cvec_ref, wsc_ref, wt_ref,
                  ach_ref, wdm_ref, whot_ref, out_ref):
    f32 = jnp.float32
    m = _NB * _N
    y = jnp.dot(sc_ref[...].reshape(m, _NTOK), wsc_ref[...],
                preferred_element_type=f32)
    y = y + jnp.dot(tors_ref[...].reshape(m, 14), wt_ref[...],
                    preferred_element_type=f32)
    iota2 = lax.broadcasted_iota(jnp.int32, (_NB, _N, _NAA + 100),
                                 2).astype(f32)
    aa3 = aa_ref[...][..., None]
    ch3 = ch_ref[...][..., None]
    twohot = ((aa3 == iota2).astype(f32)
              + (ch3 + float(_NAA) == iota2).astype(f32))
    y = y + jnp.dot(twohot.reshape(m, _NAA + 100), ach_ref[...],
                    preferred_element_type=f32)
    y3 = y.reshape(_NB, _N, _D)
    y3 = y3 + dm_ref[...][..., None] * wdm_ref[...][None]
    y3 = y3 + hot_ref[...][..., None] * whot_ref[...][None]
    y3 = y3 + mask_ref[...][..., None] * tvec_ref[...]
    y3 = y3 + (mvec_ref[...] + cvec_ref[...][None])
    out_ref[...] = y3 + y1_ref[...]


def _combine(y1, sc, tors, aa, ch, dm, hot, mask, tvec, mvec, cvec,
             wsc, wt, ach, wdm, whot):
    f32 = jnp.float32
    aux = pl.BlockSpec((_NB, _N), lambda b: (b, 0))
    return pl.pallas_call(
        _combine_body,
        grid=(_B // _NB,),
        in_specs=[
            pl.BlockSpec((_NB, _N, _D), lambda b: (b, 0, 0)),
            pl.BlockSpec((_NB, _N, _NTOK), lambda b: (b, 0, 0)),
            pl.BlockSpec((_NB, _N, 14), lambda b: (b, 0, 0)),
            aux, aux, aux, aux, aux,
            pl.BlockSpec((_NB, 1, _D), lambda b: (b, 0, 0)),
            pl.BlockSpec((_NB, 1, _D), lambda b: (b, 0, 0)),
            pl.BlockSpec((1, _D), lambda b: (0, 0)),
            pl.BlockSpec((_NTOK, _D), lambda b: (0, 0)),
            pl.BlockSpec((14, _D), lambda b: (0, 0)),
            pl.BlockSpec((_NAA + 100, _D), lambda b: (0, 0)),
            pl.BlockSpec((1, _D), lambda b: (0, 0)),
            pl.BlockSpec((1, _D), lambda b: (0, 0)),
        ],
        out_specs=pl.BlockSpec((_NB, _N, _D), lambda b: (b, 0, 0)),
        out_shape=jax.ShapeDtypeStruct((_B, _N, _D), f32),
    )(y1, sc, tors, aa, ch, dm, hot, mask, tvec, mvec, cvec,
      wsc, wt, ach, wdm, whot)


def kernel(so3_t, r3_t, cat_t, res_mask, diffuse_mask, chain_index, res_index,
           aatypes, aatypes_sc, torsions_t, structure_method, hot_spots_mask,
           aatype_table, torsion_W, torsion_b, method_table, lin_W, lin_b):
    f32 = jnp.float32
    i32 = jnp.int32

    offs = {}
    cur = 0
    for name, w in [("pos", _CPOS), ("dm", 1), ("so3", 128), ("r3", 128),
                    ("aa", _D), ("cat", 128), ("sc", _NTOK), ("chain", _CPOS),
                    ("tors", 128), ("meth", 128), ("hot", 1)]:
        offs[name] = (cur, cur + w)
        cur += w

    def wb(name):
        s, e = offs[name]
        return lin_W[s:e]

    p_tab, ch_tab, a_tab, wt_fold, tvec, mvec, cvec = _build_tables(
        so3_t.astype(f32), r3_t.astype(f32), cat_t.astype(f32),
        structure_method.astype(i32), aatype_table.astype(f32),
        torsion_W.astype(f32), torsion_b.reshape(1, 128).astype(f32),
        method_table.astype(f32),
        wb("pos"), wb("so3"), wb("r3"), wb("aa"), wb("cat"), wb("chain"),
        wb("tors"), wb("meth"), lin_b.reshape(1, _D).astype(f32))

    y1 = _sc_gather_sum(
        p_tab,
        res_index.reshape(_TOK).astype(i32),
        res_mask.reshape(_TOK).astype(f32))

    ach = jnp.concatenate([a_tab, ch_tab[:100]], axis=0)

    out = _combine(
        y1.reshape(_B, _N, _D),
        aatypes_sc.astype(f32),
        torsions_t.reshape(_B, _N, 14).astype(f32),
        aatypes.astype(f32),
        chain_index.astype(f32),
        diffuse_mask.astype(f32),
        hot_spots_mask.astype(f32),
        res_mask.astype(f32),
        tvec.reshape(_B, 1, _D),
        mvec.reshape(_B, 1, _D),
        cvec,
        wb("sc"), wt_fold, ach, wb("dm"), wb("hot"))
    return out
